# baseline (speedup 1.0000x reference)
import functools

import jax
import jax.numpy as jnp
from jax.experimental import pallas as pl
from jax.experimental.pallas import tpu as pltpu

_MIN_NORM = 1e-15
_EPS_F32 = 4e-3
_N = 4096
_D = 256
_BM = 512
_HK = _N // 2


def _artanh(v):
    v = jnp.clip(v, -1.0 + 1e-7, 1.0 - 1e-7)
    return 0.5 * (jnp.log1p(v) - jnp.log1p(-v))


def _hyp_agg_kernel(x_ref, adj0_ref, adj1_ref, o_ref, xt_ref):
    @pl.when(pl.program_id(0) == 0)
    def _compute_tangent():
        xv = x_ref[...]
        nrm = jnp.maximum(
            jnp.sqrt(jnp.sum(xv * xv, axis=1, keepdims=True)), _MIN_NORM
        )
        scale = _artanh(nrm) / nrm
        xt_ref[...] = (xv * scale).astype(jnp.bfloat16)

    a0 = adj0_ref[...].astype(jnp.bfloat16)
    a1 = adj1_ref[...].astype(jnp.bfloat16)
    s = jnp.dot(a0, xt_ref[:_HK], preferred_element_type=jnp.float32)
    s = s + jnp.dot(a1, xt_ref[_HK:], preferred_element_type=jnp.float32)
    sn = jnp.maximum(
        jnp.sqrt(jnp.sum(s * s, axis=1, keepdims=True)), _MIN_NORM
    )
    g = jnp.tanh(sn) * (s / sn)
    gn = jnp.maximum(
        jnp.sqrt(jnp.sum(g * g, axis=1, keepdims=True)), _MIN_NORM
    )
    maxnorm = 1.0 - _EPS_F32
    o_ref[...] = jnp.where(gn > maxnorm, g * (maxnorm / gn), g)


@functools.partial(jax.jit, static_argnames=())
def kernel(x, adj):
    return pl.pallas_call(
        _hyp_agg_kernel,
        grid=(_N // _BM,),
        in_specs=[
            pl.BlockSpec((_N, _D), lambda i: (0, 0)),
            pl.BlockSpec((_BM, _HK), lambda i: (i, 0)),
            pl.BlockSpec((_BM, _HK), lambda i: (i, 1)),
        ],
        out_specs=pl.BlockSpec((_BM, _D), lambda i: (i, 0)),
        out_shape=jax.ShapeDtypeStruct((_N, _D), jnp.float32),
        scratch_shapes=[pltpu.VMEM((_N, _D), jnp.bfloat16)],
    )(x, adj, adj)


# speedup vs baseline: 1.0255x; 1.0255x over previous
import functools

import jax
import jax.numpy as jnp
from jax.experimental import pallas as pl
from jax.experimental.pallas import tpu as pltpu

_MIN_NORM = 1e-15
_EPS_F32 = 4e-3
_N = 4096
_D = 256
_BM = 512
_HM = _BM // 2


def _artanh(v):
    v = jnp.clip(v, -1.0 + 1e-7, 1.0 - 1e-7)
    return 0.5 * (jnp.log1p(v) - jnp.log1p(-v))


def _postprocess(s):
    sn = jnp.maximum(
        jnp.sqrt(jnp.sum(s * s, axis=1, keepdims=True)), _MIN_NORM
    )
    g = jnp.tanh(sn) * (s / sn)
    gn = jnp.maximum(
        jnp.sqrt(jnp.sum(g * g, axis=1, keepdims=True)), _MIN_NORM
    )
    maxnorm = 1.0 - _EPS_F32
    return jnp.where(gn > maxnorm, g * (maxnorm / gn), g)


def _hyp_agg_kernel(x_ref, adj0_ref, adj1_ref, o_ref, xt_ref):
    @pl.when(pl.program_id(0) == 0)
    def _compute_tangent():
        xv = x_ref[...]
        nrm = jnp.maximum(
            jnp.sqrt(jnp.sum(xv * xv, axis=1, keepdims=True)), _MIN_NORM
        )
        scale = _artanh(nrm) / nrm
        xt_ref[...] = (xv * scale).astype(jnp.bfloat16)

    xt = xt_ref[...]
    a0 = adj0_ref[...].astype(jnp.bfloat16)
    s0 = jnp.dot(a0, xt, preferred_element_type=jnp.float32)
    o_ref[:_HM] = _postprocess(s0)
    a1 = adj1_ref[...].astype(jnp.bfloat16)
    s1 = jnp.dot(a1, xt, preferred_element_type=jnp.float32)
    o_ref[_HM:] = _postprocess(s1)


@functools.partial(jax.jit, static_argnames=())
def kernel(x, adj):
    return pl.pallas_call(
        _hyp_agg_kernel,
        grid=(_N // _BM,),
        in_specs=[
            pl.BlockSpec((_N, _D), lambda i: (0, 0)),
            pl.BlockSpec((_HM, _N), lambda i: (2 * i, 0)),
            pl.BlockSpec((_HM, _N), lambda i: (2 * i + 1, 0)),
        ],
        out_specs=pl.BlockSpec((_BM, _D), lambda i: (i, 0)),
        out_shape=jax.ShapeDtypeStruct((_N, _D), jnp.float32),
        scratch_shapes=[pltpu.VMEM((_N, _D), jnp.bfloat16)],
    )(x, adj, adj)


# speedup vs baseline: 1.1571x; 1.1283x over previous
import functools

import jax
import jax.numpy as jnp
from jax.experimental import pallas as pl
from jax.experimental.pallas import tpu as pltpu

_MIN_NORM = 1e-15
_EPS_F32 = 4e-3
_N = 4096
_D = 256
_BS = 256
_NBLK = _N // _BS
_NBUF = 10


def _artanh(v):
    v = jnp.clip(v, -1.0 + 1e-7, 1.0 - 1e-7)
    return 0.5 * (jnp.log1p(v) - jnp.log1p(-v))


def _postprocess(s):
    sn = jnp.maximum(
        jnp.sqrt(jnp.sum(s * s, axis=1, keepdims=True)), _MIN_NORM
    )
    g = jnp.tanh(sn) * (s / sn)
    gn = jnp.maximum(
        jnp.sqrt(jnp.sum(g * g, axis=1, keepdims=True)), _MIN_NORM
    )
    maxnorm = 1.0 - _EPS_F32
    return jnp.where(gn > maxnorm, g * (maxnorm / gn), g)


def _hyp_agg_kernel(x_ref, adj_ref, o_ref, xt_ref, bufs, sems):
    i = pl.program_id(0)

    def _copy(blk, slot):
        return pltpu.make_async_copy(
            adj_ref.at[pl.ds(blk * _BS, _BS), :],
            bufs.at[slot],
            sems.at[slot],
        )

    @pl.when(i == 0)
    def _prologue():
        for k in range(min(_NBUF, _NBLK)):
            _copy(k, k).start()
        xv = x_ref[...]
        nrm = jnp.maximum(
            jnp.sqrt(jnp.sum(xv * xv, axis=1, keepdims=True)), _MIN_NORM
        )
        scale = _artanh(nrm) / nrm
        xt_ref[...] = (xv * scale).astype(jnp.bfloat16)

    slot = jax.lax.rem(i, _NBUF)
    _copy(i, slot).wait()
    a = bufs[slot].astype(jnp.bfloat16)
    s = jnp.dot(a, xt_ref[...], preferred_element_type=jnp.float32)
    o_ref[...] = _postprocess(s)

    @pl.when(i + _NBUF < _NBLK)
    def _refill():
        _copy(i + _NBUF, slot).start()


@functools.partial(jax.jit, static_argnames=())
def kernel(x, adj):
    return pl.pallas_call(
        _hyp_agg_kernel,
        grid=(_NBLK,),
        in_specs=[
            pl.BlockSpec((_N, _D), lambda i: (0, 0)),
            pl.BlockSpec(memory_space=pl.ANY),
        ],
        out_specs=pl.BlockSpec((_BS, _D), lambda i: (i, 0)),
        out_shape=jax.ShapeDtypeStruct((_N, _D), jnp.float32),
        scratch_shapes=[
            pltpu.VMEM((_N, _D), jnp.bfloat16),
            pltpu.VMEM((_NBUF, _BS, _N), jnp.float32),
            pltpu.SemaphoreType.DMA((_NBUF,)),
        ],
    )(x, adj)
